# baseline (speedup 1.0000x reference)
import functools

import jax
import jax.numpy as jnp
from jax import lax
from jax.experimental import pallas as pl
from jax.experimental.pallas import tpu as pltpu
from jax.experimental.pallas import tpu_sc as plsc

DIM = 64
BATCH = 16384
CHUNK = 128

_info = plsc.get_sparse_core_info()
NC, NS, L = _info.num_cores, _info.num_subcores, _info.num_lanes
NW = NC * NS
B_PER_W = BATCH // NW
NCHUNK = B_PER_W // CHUNK
NGROUP = B_PER_W // L


def _make_sc_kernel():
    mesh = plsc.VectorSubcoreMesh(core_axis_name="c", subcore_axis_name="s")

    @functools.partial(
        pl.kernel,
        mesh=mesh,
        out_type=jax.ShapeDtypeStruct((BATCH,), jnp.float32),
        scratch_types=[
            pltpu.VMEM((NCHUNK, CHUNK), jnp.int32),
            pltpu.VMEM((NCHUNK, CHUNK), jnp.int32),
            pltpu.VMEM((B_PER_W, DIM), jnp.float32),
            pltpu.VMEM((B_PER_W, DIM), jnp.float32),
            pltpu.VMEM((B_PER_W,), jnp.float32),
            pltpu.SemaphoreType.DMA,
        ],
        compiler_params=pltpu.CompilerParams(
            needs_layout_passes=False, use_tc_tiling_on_sc=False),
    )
    def two_tower(u_hbm, i_hbm, ue_hbm, ie_hbm, out_hbm,
                  idx_u, idx_i, rows_u, rows_i, out_v, sem):
        wid = lax.axis_index("s") * NC + lax.axis_index("c")
        base = wid * B_PER_W

        pltpu.sync_copy(u_hbm.at[wid], idx_u)
        pltpu.sync_copy(i_hbm.at[wid], idx_i)

        copies = []
        for j in range(NCHUNK):
            copies.append(pltpu.async_copy(
                ue_hbm.at[idx_u.at[j]], rows_u.at[pl.ds(j * CHUNK, CHUNK)], sem))
            copies.append(pltpu.async_copy(
                ie_hbm.at[idx_i.at[j]], rows_i.at[pl.ds(j * CHUNK, CHUNK)], sem))
        for c in copies:
            c.wait()

        lane = lax.iota(jnp.int32, L)

        def group_body(g, _):
            row = g * L + lane

            def d_body(d, acc):
                col = jnp.zeros((L,), jnp.int32) + d
                ug = plsc.load_gather(rows_u, [row, col])
                ig = plsc.load_gather(rows_i, [row, col])
                return acc + ug * ig

            acc = lax.fori_loop(0, DIM, d_body,
                                jnp.zeros((L,), jnp.float32), unroll=8)
            out_v[pl.ds(g * L, L)] = acc
            return 0

        lax.fori_loop(0, NGROUP, group_body, 0)

        pltpu.sync_copy(out_v, out_hbm.at[pl.ds(base, B_PER_W)])

    return two_tower


_sc_kernel = _make_sc_kernel()


def kernel(u, i, user_emb, item_emb):
    u3 = u.astype(jnp.int32).reshape(NW, NCHUNK, CHUNK)
    i3 = i.astype(jnp.int32).reshape(NW, NCHUNK, CHUNK)
    return _sc_kernel(u3, i3, user_emb, item_emb)
